# baseline (speedup 1.0000x reference)
import jax
import jax.numpy as jnp
from jax import lax
from jax.experimental import pallas as pl
from jax.experimental.pallas import tpu as pltpu
from jax.experimental.pallas import tpu_sc as plsc

SEQ = 1569
SEQP = 1600
H = 16
HG = 8
NC = 2
NS = 16
L = 16
RPW = 104
G = 8
NCHUNK = RPW // G
NBAND = (SEQ + G - 1) // G
NCT = (SEQ + 127) // 128
NJ = NCT * 128
NJV = NJ // L
BAND = NCT * G * 128


def _sc_bias_body(tab_hbm, rpi_hbm, from_hbm, to_hbm, out_hbm,
                  tab_v, to_v, fidx_v, rows_v, pidx_v, out_v,
                  sem_in, sem_out):
    g = lax.axis_index("c")
    r = lax.axis_index("s")
    n_i = jnp.minimum(RPW, SEQ - r * RPW)
    h0 = g * HG

    def gather_rows(k):
        pltpu.sync_copy(from_hbm.at[r, k], fidx_v)
        fvec = fidx_v[...]
        for b in range(G):
            off = pl.multiple_of(fvec[b] * SEQP, 8)
            pltpu.async_copy(rpi_hbm.at[pl.ds(off, SEQP)],
                             rows_v.at[b], sem_in)

    def drain_rows():
        for b in range(G):
            pltpu.make_async_copy(rpi_hbm.at[pl.ds(0, SEQP)],
                                  rows_v.at[b], sem_in).wait()

    gather_rows(0)
    pltpu.sync_copy(tab_hbm.at[g], tab_v)
    pltpu.sync_copy(to_hbm, to_v)

    def chunk_body(k, carry):
        @pl.when(k * G < n_i)
        def _():
            band = r * NCHUNK + k
            drain_rows()
            for b in range(G):
                @plsc.parallel_loop(0, NJV, unroll=4)
                def permute(jv):
                    tvec = to_v[pl.ds(jv * L, L)]
                    rvec = plsc.load_gather(rows_v.at[b], [tvec])
                    pidx_v[b, pl.ds(jv * L, L)] = rvec * HG
            @pl.when((k + 1) * G < n_i)
            def _():
                gather_rows(k + 1)
            for h in range(HG):
                for b in range(G):
                    @plsc.parallel_loop(0, NJV, unroll=4)
                    def heads(jv):
                        base = pidx_v[b, pl.ds(jv * L, L)]
                        off = (jv // G) * 1024 + b * 128 + (jv % G) * L
                        out_v[pl.ds(off, L)] = plsc.load_gather(
                            tab_v, [base + h])
                pltpu.async_copy(
                    out_v,
                    out_hbm.at[pl.ds(((h0 + h) * NBAND + band) * BAND, BAND)],
                    sem_out).wait()
        return carry

    lax.fori_loop(0, NCHUNK, chunk_body, 0)


def kernel(relative_position_bias_table, relative_position_index, from_idx, to_idx):
    tab = relative_position_bias_table.astype(jnp.float32)
    nrel = tab.shape[0]
    tabf = jnp.stack([tab[:, :HG].reshape(-1), tab[:, HG:].reshape(-1)])
    rpi = jnp.pad(relative_position_index.astype(jnp.int32),
                  ((0, 0), (0, SEQP - SEQ))).reshape(-1)
    fi = from_idx.astype(jnp.int32)
    base3d = jnp.pad(fi, (0, NS * RPW - SEQ)).reshape(NS, NCHUNK, G)
    tail_chunk = jnp.broadcast_to(fi[SEQ - 1:], (1, G))
    row15 = jnp.concatenate(
        [base3d[NS - 1, :1], tail_chunk, base3d[NS - 1, 2:]], axis=0)
    from2d = jnp.concatenate([base3d[:NS - 1], row15[None]], axis=0)
    from2d = jnp.pad(from2d, ((0, 0), (0, 0), (0, L - G)))
    to_pad = jnp.pad(to_idx.astype(jnp.int32), (0, NJ - SEQ))
    mesh = plsc.VectorSubcoreMesh(core_axis_name="c", subcore_axis_name="s",
                                  num_cores=NC, num_subcores=NS)
    f = pl.kernel(
        _sc_bias_body,
        out_type=jax.ShapeDtypeStruct((H * NBAND * BAND,), jnp.float32),
        mesh=mesh,
        compiler_params=pltpu.CompilerParams(use_tc_tiling_on_sc=False,
                                             needs_layout_passes=False),
        scratch_types=[
            pltpu.VMEM((nrel * HG,), jnp.float32),
            pltpu.VMEM((NJ,), jnp.int32),
            pltpu.VMEM((L,), jnp.int32),
            pltpu.VMEM((G, SEQP), jnp.int32),
            pltpu.VMEM((G, NJ), jnp.int32),
            pltpu.VMEM((BAND,), jnp.float32),
            pltpu.SemaphoreType.DMA,
            pltpu.SemaphoreType.DMA,
        ],
    )
    flat = f(tabf, rpi, from2d, to_pad)
    out5 = flat.reshape(H, NBAND, NCT, G, 128)
    return (out5.transpose(0, 1, 3, 2, 4)
            .reshape(H, NBAND * G, NCT * 128)[:, :SEQ, :SEQ])


# speedup vs baseline: 1.0500x; 1.0500x over previous
import jax
import jax.numpy as jnp
from jax import lax
from jax.experimental import pallas as pl
from jax.experimental.pallas import tpu as pltpu
from jax.experimental.pallas import tpu_sc as plsc

SEQ = 1569
SEQP = 1600
H = 16
HG = 8
NC = 2
NS = 16
L = 16
RPW = 104
G = 8
NCHUNK = RPW // G
NBAND = (SEQ + G - 1) // G
NCT = (SEQ + 127) // 128
NJ = NCT * 128
NJV = NJ // L
BAND = NCT * G * 128


def _sc_bias_body(tab_hbm, rpi_hbm, from_hbm, to_hbm, out_hbm,
                  tab_v, to_v, fidx_v, rows_v, pidx_v, out_a, out_b,
                  sem_in, sem_out):
    g = lax.axis_index("c")
    r = lax.axis_index("s")
    n_i = jnp.minimum(RPW, SEQ - r * RPW)
    h0 = g * HG

    def gather_rows(k):
        pltpu.sync_copy(from_hbm.at[r, k], fidx_v)
        fvec = fidx_v[...]
        for b in range(G):
            off = pl.multiple_of(fvec[b] * SEQP, 8)
            pltpu.async_copy(rpi_hbm.at[pl.ds(off, SEQP)],
                             rows_v.at[b], sem_in)

    def drain_rows():
        for b in range(G):
            pltpu.make_async_copy(rpi_hbm.at[pl.ds(0, SEQP)],
                                  rows_v.at[b], sem_in).wait()

    gather_rows(0)
    pltpu.sync_copy(tab_hbm.at[g], tab_v)
    pltpu.sync_copy(to_hbm, to_v)

    def chunk_body(k, carry):
        @pl.when(k * G < n_i)
        def _():
            band = r * NCHUNK + k
            drain_rows()
            for b in range(G):
                @plsc.parallel_loop(0, NJV, unroll=4)
                def permute(jv):
                    tvec = to_v[pl.ds(jv * L, L)]
                    rvec = plsc.load_gather(rows_v.at[b], [tvec])
                    pidx_v[b, pl.ds(jv * L, L)] = rvec * HG
            @pl.when((k + 1) * G < n_i)
            def _():
                gather_rows(k + 1)
            pend = [None, None]
            for h in range(HG):
                hbase = ((h0 + h) * NBAND + band) * BAND
                if pend[0] is not None:
                    pend[0].wait()
                for b in range(G):
                    @plsc.parallel_loop(0, 6 * G, unroll=4)
                    def heads_a(jv):
                        base = pidx_v[b, pl.ds(jv * L, L)]
                        off = (jv // G) * 1024 + b * 128 + (jv % G) * L
                        out_a[pl.ds(off, L)] = plsc.load_gather(
                            tab_v, [base + h])
                pend[0] = pltpu.async_copy(
                    out_a, out_hbm.at[pl.ds(hbase, 6 * 1024)], sem_out)
                if pend[1] is not None:
                    pend[1].wait()
                for b in range(G):
                    @plsc.parallel_loop(6 * G, NJV, unroll=4)
                    def heads_b(jv):
                        base = pidx_v[b, pl.ds(jv * L, L)]
                        off = (jv // G - 6) * 1024 + b * 128 + (jv % G) * L
                        out_b[pl.ds(off, L)] = plsc.load_gather(
                            tab_v, [base + h])
                pend[1] = pltpu.async_copy(
                    out_b, out_hbm.at[pl.ds(hbase + 6 * 1024, 7 * 1024)],
                    sem_out)
            pend[0].wait()
            pend[1].wait()
        return carry

    lax.fori_loop(0, NCHUNK, chunk_body, 0)


def kernel(relative_position_bias_table, relative_position_index, from_idx, to_idx):
    tab = relative_position_bias_table.astype(jnp.float32)
    nrel = tab.shape[0]
    tabf = jnp.stack([tab[:, :HG].reshape(-1), tab[:, HG:].reshape(-1)])
    rpi = jnp.pad(relative_position_index.astype(jnp.int32),
                  ((0, 0), (0, SEQP - SEQ))).reshape(-1)
    fi = from_idx.astype(jnp.int32)
    base3d = jnp.pad(fi, (0, NS * RPW - SEQ)).reshape(NS, NCHUNK, G)
    tail_chunk = jnp.broadcast_to(fi[SEQ - 1:], (1, G))
    row15 = jnp.concatenate(
        [base3d[NS - 1, :1], tail_chunk, base3d[NS - 1, 2:]], axis=0)
    from2d = jnp.concatenate([base3d[:NS - 1], row15[None]], axis=0)
    from2d = jnp.pad(from2d, ((0, 0), (0, 0), (0, L - G)))
    to_pad = jnp.pad(to_idx.astype(jnp.int32), (0, NJ - SEQ))
    mesh = plsc.VectorSubcoreMesh(core_axis_name="c", subcore_axis_name="s",
                                  num_cores=NC, num_subcores=NS)
    f = pl.kernel(
        _sc_bias_body,
        out_type=jax.ShapeDtypeStruct((H * NBAND * BAND,), jnp.float32),
        mesh=mesh,
        compiler_params=pltpu.CompilerParams(use_tc_tiling_on_sc=False,
                                             needs_layout_passes=False),
        scratch_types=[
            pltpu.VMEM((nrel * HG,), jnp.float32),
            pltpu.VMEM((NJ,), jnp.int32),
            pltpu.VMEM((L,), jnp.int32),
            pltpu.VMEM((G, SEQP), jnp.int32),
            pltpu.VMEM((G, NJ), jnp.int32),
            pltpu.VMEM((6 * 1024,), jnp.float32),
            pltpu.VMEM((7 * 1024,), jnp.float32),
            pltpu.SemaphoreType.DMA,
            pltpu.SemaphoreType.DMA,
        ],
    )
    flat = f(tabf, rpi, from2d, to_pad)
    out5 = flat.reshape(H, NBAND, NCT, G, 128)
    return (out5.transpose(0, 1, 3, 2, 4)
            .reshape(H, NBAND * G, NCT * 128)[:, :SEQ, :SEQ])


# speedup vs baseline: 1.0539x; 1.0037x over previous
import jax
import jax.numpy as jnp
from jax import lax
from jax.experimental import pallas as pl
from jax.experimental.pallas import tpu as pltpu
from jax.experimental.pallas import tpu_sc as plsc

SEQ = 1569
SEQP = 1600
H = 16
HG = 8
NC = 2
NS = 16
L = 16
RPW = 104
G = 8
NCHUNK = RPW // G
NBAND = (SEQ + G - 1) // G
NCT = (SEQ + 127) // 128
NJ = NCT * 128
NJV = NJ // L
BAND = NCT * G * 128


def _sc_bias_body(tab_hbm, rpi_hbm, from_hbm, to_hbm, out_hbm,
                  tab_v, to_v, fidx_v, rows_v, pidx_v, out_a, out_b,
                  sem_in, sem_out):
    g = lax.axis_index("c")
    r = lax.axis_index("s")
    n_i = jnp.minimum(RPW, SEQ - r * RPW)
    h0 = g * HG

    def gather_rows(k):
        pltpu.sync_copy(
            from_hbm.at[pl.ds((r * NCHUNK + k) * L, L)], fidx_v)
        fvec = fidx_v[...]
        for b in range(G):
            off = pl.multiple_of(fvec[b] * SEQP, 8)
            pltpu.async_copy(rpi_hbm.at[pl.ds(off, SEQP)],
                             rows_v.at[b], sem_in)

    def drain_rows():
        for b in range(G):
            pltpu.make_async_copy(rpi_hbm.at[pl.ds(0, SEQP)],
                                  rows_v.at[b], sem_in).wait()

    gather_rows(0)
    toff = pl.multiple_of(g * (tab_hbm.shape[0] // NC), 8)
    pltpu.sync_copy(tab_hbm.at[pl.ds(toff, tab_hbm.shape[0] // NC)], tab_v)
    pltpu.sync_copy(to_hbm, to_v)

    def chunk_body(k, carry):
        @pl.when(k * G < n_i)
        def _():
            band = r * NCHUNK + k
            drain_rows()
            for b in range(G):
                @plsc.parallel_loop(0, NJV, unroll=4)
                def permute(jv):
                    tvec = to_v[pl.ds(jv * L, L)]
                    rvec = plsc.load_gather(rows_v.at[b], [tvec])
                    pidx_v[b, pl.ds(jv * L, L)] = rvec * HG
            @pl.when((k + 1) * G < n_i)
            def _():
                gather_rows(k + 1)
            pend = [None, None]
            for h in range(HG):
                hbase = ((h0 + h) * NBAND + band) * BAND
                if pend[0] is not None:
                    pend[0].wait()
                for b in range(G):
                    @plsc.parallel_loop(0, 6 * G, unroll=4)
                    def heads_a(jv):
                        base = pidx_v[b, pl.ds(jv * L, L)]
                        off = (jv // G) * 1024 + b * 128 + (jv % G) * L
                        out_a[pl.ds(off, L)] = plsc.load_gather(
                            tab_v, [base + h])
                pend[0] = pltpu.async_copy(
                    out_a, out_hbm.at[pl.ds(hbase, 6 * 1024)], sem_out)
                if pend[1] is not None:
                    pend[1].wait()
                for b in range(G):
                    @plsc.parallel_loop(6 * G, NJV, unroll=4)
                    def heads_b(jv):
                        base = pidx_v[b, pl.ds(jv * L, L)]
                        off = (jv // G - 6) * 1024 + b * 128 + (jv % G) * L
                        out_b[pl.ds(off, L)] = plsc.load_gather(
                            tab_v, [base + h])
                pend[1] = pltpu.async_copy(
                    out_b, out_hbm.at[pl.ds(hbase + 6 * 1024, 7 * 1024)],
                    sem_out)
            pend[0].wait()
            pend[1].wait()
        return carry

    lax.fori_loop(0, NCHUNK, chunk_body, 0)


def kernel(relative_position_bias_table, relative_position_index, from_idx, to_idx):
    tab = relative_position_bias_table.astype(jnp.float32)
    nrel = tab.shape[0]
    tabf = jnp.concatenate([tab[:, :HG].reshape(-1), tab[:, HG:].reshape(-1)])
    rpi = jnp.pad(relative_position_index.astype(jnp.int32),
                  ((0, 0), (0, SEQP - SEQ))).reshape(-1)
    fi = from_idx.astype(jnp.int32)
    base3d = jnp.pad(fi, (0, NS * RPW - SEQ)).reshape(NS, NCHUNK, G)
    tail_chunk = jnp.broadcast_to(fi[SEQ - 1:], (1, G))
    row15 = jnp.concatenate(
        [base3d[NS - 1, :1], tail_chunk, base3d[NS - 1, 2:]], axis=0)
    from2d = jnp.concatenate([base3d[:NS - 1], row15[None]], axis=0)
    from2d = jnp.pad(from2d, ((0, 0), (0, 0), (0, L - G))).reshape(-1)
    to_pad = jnp.pad(to_idx.astype(jnp.int32), (0, NJ - SEQ))
    mesh = plsc.VectorSubcoreMesh(core_axis_name="c", subcore_axis_name="s",
                                  num_cores=NC, num_subcores=NS)
    f = pl.kernel(
        _sc_bias_body,
        out_type=jax.ShapeDtypeStruct((H * NBAND * BAND,), jnp.float32),
        mesh=mesh,
        compiler_params=pltpu.CompilerParams(use_tc_tiling_on_sc=False,
                                             needs_layout_passes=False),
        scratch_types=[
            pltpu.VMEM((nrel * HG,), jnp.float32),
            pltpu.VMEM((NJ,), jnp.int32),
            pltpu.VMEM((L,), jnp.int32),
            pltpu.VMEM((G, SEQP), jnp.int32),
            pltpu.VMEM((G, NJ), jnp.int32),
            pltpu.VMEM((6 * 1024,), jnp.float32),
            pltpu.VMEM((7 * 1024,), jnp.float32),
            pltpu.SemaphoreType.DMA,
            pltpu.SemaphoreType.DMA,
        ],
    )
    flat = f(tabf, rpi, from2d, to_pad)
    out5 = flat.reshape(H, NBAND, NCT, G, 128)
    return (out5.transpose(0, 1, 3, 2, 4)
            .reshape(H, NBAND * G, NCT * 128)[:, :SEQ, :SEQ])
